# baseline (speedup 1.0000x reference)
import functools

import jax
import jax.numpy as jnp
from jax import lax
from jax.experimental import pallas as pl
from jax.experimental.pallas import tpu as pltpu
from jax.experimental.pallas import tpu_sc as plsc

B, L, D = 4096, 200, 64
DP = 128
N = B * L
CMAX = 104
OFF = (0, 104)
SZ = (104, 96)
NBUF = 2
LANES = 16


def _sc_body(s_per_w, item_idx, test_idx, tag_idx,
             item_tab, test_tab, tag_tab, pos_tab, out,
             idx_v, rows_v, stage_v, pos_v,
             isem0, isem1, gsem0, gsem1, osem):
    nc = plsc.get_sparse_core_info().num_cores
    wid = lax.axis_index("s") * nc + lax.axis_index("c")
    seq0 = wid * s_per_w
    isems = (isem0, isem1)
    gsems = (gsem0, gsem1)
    idx_hbms = (item_idx, test_idx, tag_idx)
    tabs = (item_tab, test_tab, tag_tab)

    pltpu.sync_copy(pos_tab, pos_v)

    def islot(j, t):
        return pl.ds((j * 3 + t) * CMAX, SZ[j])

    def issue_idx(seq, j):
        base = seq * L + OFF[j]
        for t in range(3):
            pltpu.async_copy(idx_hbms[t].at[pl.ds(base, SZ[j])],
                             idx_v.at[islot(j, t)], isems[j])

    def wait_idx(j):
        for t in range(3):
            pltpu.make_async_copy(idx_hbms[t].at[pl.ds(0, SZ[j])],
                                  idx_v.at[islot(j, t)], isems[j]).wait()

    def issue_gathers(j):
        for t in range(3):
            pltpu.async_copy(tabs[t].at[idx_v.at[islot(j, t)]],
                             rows_v.at[j, t, pl.ds(0, SZ[j])], gsems[j])

    def wait_gathers(j):
        for t in range(3):
            pltpu.make_async_copy(tabs[t].at[idx_v.at[islot(j, t)]],
                                  rows_v.at[j, t, pl.ds(0, SZ[j])],
                                  gsems[j]).wait()

    def issue_out(seq, j):
        pltpu.async_copy(stage_v.at[pl.ds(0, SZ[j])],
                         out.at[seq, pl.ds(OFF[j], SZ[j])], osem)

    def wait_out(j):
        pltpu.make_async_copy(stage_v.at[pl.ds(0, SZ[j])],
                              out.at[0, pl.ds(OFF[j], SZ[j])], osem).wait()

    def compute(j):
        ita = rows_v.at[j, 0]
        tst = rows_v.at[j, 1]
        tag = rows_v.at[j, 2]

        def row(i, carry):
            p = OFF[j] + i
            for q in range(D // LANES):
                sl = pl.ds(q * LANES, LANES)
                pv = pos_v[pl.ds(p * D + q * LANES, LANES)]
                stage_v[i, sl] = (ita[i, sl] + tst[i, sl] + tag[i, sl]
                                  + pv) * 0.25
            return carry

        lax.fori_loop(0, SZ[j], row, 0, unroll=2)

    issue_idx(seq0, 0)
    issue_idx(seq0, 1)
    wait_idx(0)
    issue_gathers(0)

    def step(m, carry):
        seq = seq0 + m
        for j in range(NBUF):
            nj = (j + 1) % NBUF
            wait_gathers(j)

            @pl.when(m + 1 < s_per_w)
            def _():
                issue_idx(seq + 1, j)

            @pl.when((j == 0) | (m + 1 < s_per_w))
            def _():
                wait_idx(nj)
                issue_gathers(nj)

            @pl.when((m > 0) | (j > 0))
            def _():
                wait_out(nj)

            compute(j)
            issue_out(seq, j)
        return carry

    lax.fori_loop(0, s_per_w, step, 0, unroll=False)
    wait_out(1)


def kernel(item_idx, test_idx, tag_idx, item_table, test_table, tag_table,
           pos_table):
    info = plsc.get_sparse_core_info()
    nw = info.num_cores * info.num_subcores
    s_per_w = B // nw

    item2 = item_idx.astype(jnp.int32).reshape(N)
    test2 = test_idx.astype(jnp.int32).reshape(N)
    tag2 = tag_idx.astype(jnp.int32).reshape(N)
    pad = ((0, 0), (0, DP - D))
    itab = jnp.pad(item_table, pad)
    ttab = jnp.pad(test_table, pad)
    gtab = jnp.pad(tag_table, pad)
    pos1 = pos_table.reshape(L * D)

    mesh = plsc.VectorSubcoreMesh(core_axis_name="c", subcore_axis_name="s")
    run = functools.partial(
        pl.kernel,
        out_type=jax.ShapeDtypeStruct((B, L, D), jnp.float32),
        mesh=mesh,
        compiler_params=pltpu.CompilerParams(use_tc_tiling_on_sc=True),
        scratch_types=[
            pltpu.VMEM((NBUF * 3 * CMAX,), jnp.int32),
            pltpu.VMEM((NBUF, 3, CMAX, DP), jnp.float32),
            pltpu.VMEM((CMAX, D), jnp.float32),
            pltpu.VMEM((L * D,), jnp.float32),
            pltpu.SemaphoreType.DMA,
            pltpu.SemaphoreType.DMA,
            pltpu.SemaphoreType.DMA,
            pltpu.SemaphoreType.DMA,
            pltpu.SemaphoreType.DMA,
        ],
    )(functools.partial(_sc_body, s_per_w))

    return run(item2, test2, tag2, itab, ttab, gtab, pos1)


# speedup vs baseline: 1.0500x; 1.0500x over previous
import functools

import jax
import jax.numpy as jnp
from jax import lax
from jax.experimental import pallas as pl
from jax.experimental.pallas import tpu as pltpu
from jax.experimental.pallas import tpu_sc as plsc

B, L, D = 4096, 200, 64
DP = 128
N = B * L
C = 128
NBUF = 2
LANES = 16


def _sc_body(g_per_w, item_idx, test_idx, tag_idx,
             item_tab, test_tab, tag_tab, pos_tab, out,
             idx_v, rows_v, stage_v, pos_v,
             isem0, isem1, gsem0, gsem1, osem):
    nc = plsc.get_sparse_core_info().num_cores
    wid = lax.axis_index("s") * nc + lax.axis_index("c")
    row0 = wid * g_per_w
    isems = (isem0, isem1)
    gsems = (gsem0, gsem1)
    idx_hbms = (item_idx, test_idx, tag_idx)
    tabs = (item_tab, test_tab, tag_tab)

    pltpu.sync_copy(pos_tab, pos_v)

    def islot(b, t):
        return pl.ds((b * 3 + t) * C, C)

    def issue_idx(g, b):
        base = (row0 + g) * C
        for t in range(3):
            pltpu.async_copy(idx_hbms[t].at[pl.ds(base, C)],
                             idx_v.at[islot(b, t)], isems[b])

    def wait_idx(b):
        for t in range(3):
            pltpu.make_async_copy(idx_hbms[t].at[pl.ds(0, C)],
                                  idx_v.at[islot(b, t)], isems[b]).wait()

    def issue_gathers(b):
        for t in range(3):
            pltpu.async_copy(tabs[t].at[idx_v.at[islot(b, t)]],
                             rows_v.at[b, t], gsems[b])

    def wait_gathers(b):
        for t in range(3):
            pltpu.make_async_copy(tabs[t].at[idx_v.at[islot(b, t)]],
                                  rows_v.at[b, t], gsems[b]).wait()

    def issue_out(g):
        base = (row0 + g) * C
        pltpu.async_copy(stage_v, out.at[pl.ds(base, C)], osem)

    def wait_out():
        pltpu.make_async_copy(stage_v, out.at[pl.ds(0, C)], osem).wait()

    def compute(g, b):
        ita = rows_v.at[b, 0]
        tst = rows_v.at[b, 1]
        tag = rows_v.at[b, 2]
        pbase = lax.rem((row0 + g) * C, L)

        def row(i, p):
            for q in range(D // LANES):
                sl = pl.ds(q * LANES, LANES)
                pv = pos_v[pl.ds(p * D + q * LANES, LANES)]
                stage_v[i, sl] = (ita[i, sl] + tst[i, sl] + tag[i, sl]
                                  + pv) * 0.25
            p = p + 1
            return lax.select(p == L, 0, p)

        lax.fori_loop(0, C, row, pbase, unroll=2)

    issue_idx(0, 0)
    issue_idx(1, 1)
    wait_idx(0)
    issue_gathers(0)

    def step(m, carry):
        for j in range(NBUF):
            g = m * NBUF + j
            nb = (j + 1) % NBUF
            wait_gathers(j)

            @pl.when(g + 2 < g_per_w)
            def _():
                issue_idx(g + 2, j)

            @pl.when(g + 1 < g_per_w)
            def _():
                wait_idx(nb)
                issue_gathers(nb)

            @pl.when(g > 0)
            def _():
                wait_out()

            compute(g, j)
            issue_out(g)
        return carry

    lax.fori_loop(0, g_per_w // NBUF, step, 0, unroll=False)
    wait_out()


def _pad_body(src, dst):
    blk = src[...]
    dst[...] = jnp.concatenate([blk, jnp.zeros_like(blk)], axis=1)


def _pad128(tab):
    v = tab.shape[0]
    bs = 2048
    return pl.pallas_call(
        _pad_body,
        grid=(pl.cdiv(v, bs),),
        in_specs=[pl.BlockSpec((bs, D), lambda i: (i, 0))],
        out_specs=pl.BlockSpec((bs, DP), lambda i: (i, 0)),
        out_shape=jax.ShapeDtypeStruct((v, DP), jnp.float32),
    )(tab)


def kernel(item_idx, test_idx, tag_idx, item_table, test_table, tag_table,
           pos_table):
    info = plsc.get_sparse_core_info()
    nw = info.num_cores * info.num_subcores
    g_per_w = N // (C * nw)

    item2 = item_idx.astype(jnp.int32).reshape(N)
    test2 = test_idx.astype(jnp.int32).reshape(N)
    tag2 = tag_idx.astype(jnp.int32).reshape(N)
    itab = _pad128(item_table)
    ttab = _pad128(test_table)
    gtab = _pad128(tag_table)
    pos1 = pos_table.reshape(L * D)

    mesh = plsc.VectorSubcoreMesh(core_axis_name="c", subcore_axis_name="s")
    run = functools.partial(
        pl.kernel,
        out_type=jax.ShapeDtypeStruct((N, D), jnp.float32),
        mesh=mesh,
        compiler_params=pltpu.CompilerParams(use_tc_tiling_on_sc=True),
        scratch_types=[
            pltpu.VMEM((NBUF * 3 * C,), jnp.int32),
            pltpu.VMEM((NBUF, 3, C, DP), jnp.float32),
            pltpu.VMEM((C, D), jnp.float32),
            pltpu.VMEM((L * D,), jnp.float32),
            pltpu.SemaphoreType.DMA,
            pltpu.SemaphoreType.DMA,
            pltpu.SemaphoreType.DMA,
            pltpu.SemaphoreType.DMA,
            pltpu.SemaphoreType.DMA,
        ],
    )(functools.partial(_sc_body, g_per_w))

    out = run(item2, test2, tag2, itab, ttab, gtab, pos1)
    return out.reshape(B, L, D)


# speedup vs baseline: 1.0817x; 1.0302x over previous
import functools

import jax
import jax.numpy as jnp
from jax import lax
from jax.experimental import pallas as pl
from jax.experimental.pallas import tpu as pltpu
from jax.experimental.pallas import tpu_sc as plsc

B, L, D = 4096, 200, 64
DP = 128
N = B * L
C = 128
NBUF = 2
LANES = 16


def _sc_body(g_per_w, item_idx, test_idx, tag_idx,
             item_tab, test_tab, tag_tab, pos_tab, out,
             idx_v, rows_v, stage_v, pos_v,
             isem0, isem1, gsem0, gsem1, osem):
    nc = plsc.get_sparse_core_info().num_cores
    wid = lax.axis_index("s") * nc + lax.axis_index("c")
    row0 = wid * g_per_w
    isems = (isem0, isem1)
    gsems = (gsem0, gsem1)
    idx_hbms = (item_idx, test_idx, tag_idx)
    tabs = (item_tab, test_tab, tag_tab)

    pltpu.sync_copy(pos_tab, pos_v)

    def islot(b, t):
        return pl.ds((b * 3 + t) * C, C)

    def issue_idx(g, b):
        base = (row0 + g) * C
        for t in range(3):
            pltpu.async_copy(idx_hbms[t].at[pl.ds(base, C)],
                             idx_v.at[islot(b, t)], isems[b])

    def wait_idx(b):
        for t in range(3):
            pltpu.make_async_copy(idx_hbms[t].at[pl.ds(0, C)],
                                  idx_v.at[islot(b, t)], isems[b]).wait()

    def issue_gathers(b):
        for t in range(3):
            pltpu.async_copy(tabs[t].at[idx_v.at[islot(b, t)]],
                             rows_v.at[b, t], gsems[b])

    def wait_gathers(b):
        for t in range(3):
            pltpu.make_async_copy(tabs[t].at[idx_v.at[islot(b, t)]],
                                  rows_v.at[b, t], gsems[b]).wait()

    def issue_out(g):
        base = (row0 + g) * C
        pltpu.async_copy(stage_v, out.at[pl.ds(base, C)], osem)

    def wait_out():
        pltpu.make_async_copy(stage_v, out.at[pl.ds(0, C)], osem).wait()

    def compute(g, b):
        ita = rows_v.at[b, 0]
        tst = rows_v.at[b, 1]
        tag = rows_v.at[b, 2]
        pbase = lax.rem((row0 + g) * C, L)

        def row(i, p):
            for q in range(D // LANES):
                sl = pl.ds(q * LANES, LANES)
                pv = pos_v[pl.ds(p * D + q * LANES, LANES)]
                stage_v[i, sl] = (ita[i, sl] + tst[i, sl] + tag[i, sl]
                                  + pv) * 0.25
            p = p + 1
            return lax.select(p == L, 0, p)

        lax.fori_loop(0, C, row, pbase, unroll=2)

    issue_idx(0, 0)
    issue_idx(1, 1)
    wait_idx(0)
    issue_gathers(0)

    def step(m, carry):
        for j in range(NBUF):
            g = m * NBUF + j
            nb = (j + 1) % NBUF
            wait_gathers(j)

            @pl.when(g + 2 < g_per_w)
            def _():
                issue_idx(g + 2, j)

            @pl.when(g + 1 < g_per_w)
            def _():
                wait_idx(nb)
                issue_gathers(nb)

            @pl.when(g > 0)
            def _():
                wait_out()

            compute(g, j)
            issue_out(g)
        return carry

    lax.fori_loop(0, g_per_w // NBUF, step, 0, unroll=False)
    wait_out()


def _pad_body3(it_s, te_s, ta_s, it_d, te_d, ta_d):
    i = pl.program_id(0)
    blk = it_s[...]
    it_d[...] = jnp.concatenate([blk, jnp.zeros_like(blk)], axis=1)

    @pl.when(i == 0)
    def _():
        tb = te_s[...]
        te_d[...] = jnp.concatenate([tb, jnp.zeros_like(tb)], axis=1)
        gb = ta_s[...]
        ta_d[...] = jnp.concatenate([gb, jnp.zeros_like(gb)], axis=1)


def _pad128x3(itab, ttab, gtab):
    vi = itab.shape[0]
    vt = ttab.shape[0]
    vg = gtab.shape[0]
    bs = 4096
    return pl.pallas_call(
        _pad_body3,
        grid=(pl.cdiv(vi, bs),),
        in_specs=[
            pl.BlockSpec((bs, D), lambda i: (i, 0)),
            pl.BlockSpec((vt, D), lambda i: (0, 0)),
            pl.BlockSpec((vg, D), lambda i: (0, 0)),
        ],
        out_specs=[
            pl.BlockSpec((bs, DP), lambda i: (i, 0)),
            pl.BlockSpec((vt, DP), lambda i: (0, 0)),
            pl.BlockSpec((vg, DP), lambda i: (0, 0)),
        ],
        out_shape=[
            jax.ShapeDtypeStruct((vi, DP), jnp.float32),
            jax.ShapeDtypeStruct((vt, DP), jnp.float32),
            jax.ShapeDtypeStruct((vg, DP), jnp.float32),
        ],
    )(itab, ttab, gtab)


def kernel(item_idx, test_idx, tag_idx, item_table, test_table, tag_table,
           pos_table):
    info = plsc.get_sparse_core_info()
    nw = info.num_cores * info.num_subcores
    g_per_w = N // (C * nw)

    item2 = item_idx.astype(jnp.int32).reshape(N)
    test2 = test_idx.astype(jnp.int32).reshape(N)
    tag2 = tag_idx.astype(jnp.int32).reshape(N)
    itab, ttab, gtab = _pad128x3(item_table, test_table, tag_table)
    pos1 = pos_table.reshape(L * D)

    mesh = plsc.VectorSubcoreMesh(core_axis_name="c", subcore_axis_name="s")
    run = functools.partial(
        pl.kernel,
        out_type=jax.ShapeDtypeStruct((N, D), jnp.float32),
        mesh=mesh,
        compiler_params=pltpu.CompilerParams(use_tc_tiling_on_sc=True),
        scratch_types=[
            pltpu.VMEM((NBUF * 3 * C,), jnp.int32),
            pltpu.VMEM((NBUF, 3, C, DP), jnp.float32),
            pltpu.VMEM((C, D), jnp.float32),
            pltpu.VMEM((L * D,), jnp.float32),
            pltpu.SemaphoreType.DMA,
            pltpu.SemaphoreType.DMA,
            pltpu.SemaphoreType.DMA,
            pltpu.SemaphoreType.DMA,
            pltpu.SemaphoreType.DMA,
        ],
    )(functools.partial(_sc_body, g_per_w))

    out = run(item2, test2, tag2, itab, ttab, gtab, pos1)
    return out.reshape(B, L, D)
